# baseline (speedup 1.0000x reference)
import functools

import jax
import jax.numpy as jnp
from jax.experimental import pallas as pl
from jax.experimental.pallas import tpu as pltpu

IN_EPS = 1e-5
NEG_SLOPE = 0.01


def _instance_norm(y, gamma, beta):
    mu = jnp.mean(y, axis=-1, keepdims=True)
    var = jnp.mean((y - mu) ** 2, axis=-1, keepdims=True)
    return (y - mu) * jax.lax.rsqrt(var + IN_EPS) * gamma + beta


def _leaky_relu(y):
    return jnp.where(y > 0, y, NEG_SLOPE * y)


def _rep(a):
    return pl.BlockSpec(a.shape, lambda b, _n=a.ndim: (0,) * _n)


def _hw_masks(seg, hlen, wlen):
    l = jax.lax.broadcasted_iota(jnp.int32, (1, seg), 1)
    hw = l % (hlen * wlen)
    hv = hw // wlen
    wv = hw % wlen
    masks = {}
    for sh in (-1, 0, 1):
        for sw in (-1, 0, 1):
            if sh == 0 and sw == 0:
                masks[(sh, sw)] = None
                continue
            valid = ((hv + sh >= 0) & (hv + sh < hlen)
                     & (wv + sw >= 0) & (wv + sw < wlen))
            masks[(sh, sw)] = valid.astype(jnp.float32)
    return masks


def _taps_to_scratch(x, p_s, masks, *, seg, pad, wlen, rows):
    segp = seg + 2 * pad
    slots = {}
    for kh in range(3):
        for kw in range(3):
            t = kh * 3 + kw
            for qh in range(2):
                for qw in range(2):
                    q = qh * 2 + qw
                    qsrc = ((qh + kh - 1) % 2) * 2 + ((qw + kw - 1) % 2)
                    sh = (qh + kh - 1) // 2
                    sw = (qw + kw - 1) // 2
                    slots.setdefault((qsrc, sh, sw), []).append((t, q))
    for (qsrc, sh, sw), dests in slots.items():
        src = x[:, qsrc * seg:(qsrc + 1) * seg]
        off = sh * wlen + sw
        if off:
            src = pltpu.roll(src, (-off) % seg, axis=1)
        m = masks[(sh, sw)]
        if m is not None:
            src = src * m
        srcb = src.astype(p_s.dtype)
        for (t, q) in dests:
            c0 = q * segp + pad
            p_s[t * rows:(t + 1) * rows, c0:c0 + seg] = srcb


def _conv3_hybrid(p_s, w_ref, *, seg, pad):
    segp = seg + 2 * pad
    wide = 3 * segp + seg
    acc = None
    for kd in range(3):
        d = jnp.dot(w_ref[kd], p_s[:, kd * pad:kd * pad + wide],
                    preferred_element_type=jnp.float32)
        acc = d if acc is None else acc + d
    return jnp.concatenate(
        [acc[:, q * segp:q * segp + seg] for q in range(4)], axis=1)


def _zero_pads(p_s, *, seg, pad):
    segp = seg + 2 * pad
    z = jnp.zeros((p_s.shape[0], pad), p_s.dtype)
    for q in range(4):
        p_s[:, q * segp:q * segp + pad] = z
        p_s[:, q * segp + pad + seg:(q + 1) * segp] = z


def _fused_body(inp_ref, skip_ref, vperm_ref, uperm_ref, wt_ref, w1_ref, g1_ref, b1_ref,
                w3_ref, g3_ref, b3_ref, w2_ref, g2_ref, b2_ref,
                o_ref, p1_s, p2_s, *, seg, pad, wlen, hlen):
    @pl.when(pl.program_id(0) == 0)
    def _():
        _zero_pads(p1_s, seg=seg, pad=pad)
        _zero_pads(p2_s, seg=seg, pad=pad)
    for blk in range(o_ref.shape[0]):
        _one_batch(blk, inp_ref, skip_ref, vperm_ref, uperm_ref, wt_ref,
                   w1_ref, g1_ref, b1_ref, w3_ref, g3_ref, b3_ref,
                   w2_ref, g2_ref, b2_ref, o_ref, p1_s, p2_s,
                   seg=seg, pad=pad, wlen=wlen, hlen=hlen)


def _one_batch(blk, inp_ref, skip_ref, vperm_ref, uperm_ref, wt_ref, w1_ref,
               g1_ref, b1_ref, w3_ref, g3_ref, b3_ref, w2_ref, g2_ref, b2_ref,
               o_ref, p1_s, p2_s, *, seg, pad, wlen, hlen):
    cout = o_ref.shape[1]
    dblk = pad
    nd = seg // dblk
    hwb = 4 * dblk

    up2 = jnp.dot(wt_ref[...], inp_ref[blk],
                  preferred_element_type=jnp.float32)

    segs = []
    for q in range(4):
        a = up2[q * cout:(q + 1) * cout]
        b = up2[(4 + q) * cout:(5 + q) * cout]
        for dp in range(nd // 2):
            segs.append(a[:, dp * dblk:(dp + 1) * dblk])
            segs.append(b[:, dp * dblk:(dp + 1) * dblk])
    x_up = jnp.concatenate(segs, axis=1)

    skipm = skip_ref[blk].reshape(cout * nd, hwb).astype(vperm_ref.dtype)
    permres = jnp.dot(skipm, vperm_ref[...],
                      preferred_element_type=jnp.float32)
    perm3 = permres.reshape(cout, nd, hwb)
    x_skip = jnp.concatenate(
        [perm3[:, d, q * dblk:(q + 1) * dblk]
         for q in range(4) for d in range(nd)], axis=1)
    x = jnp.concatenate([x_up, x_skip], axis=0)

    masks = _hw_masks(seg, hlen, wlen)

    _taps_to_scratch(x, p1_s, masks, seg=seg, pad=pad, wlen=wlen,
                     rows=2 * cout)
    y1 = _leaky_relu(_instance_norm(
        _conv3_hybrid(p1_s, w1_ref, seg=seg, pad=pad),
        g1_ref[...], b1_ref[...]))

    r = _instance_norm(
        jnp.dot(w3_ref[...], x, preferred_element_type=jnp.float32),
        g3_ref[...], b3_ref[...])

    _taps_to_scratch(y1, p2_s, masks, seg=seg, pad=pad, wlen=wlen, rows=cout)
    y2 = _instance_norm(
        _conv3_hybrid(p2_s, w2_ref, seg=seg, pad=pad),
        g2_ref[...], b2_ref[...])
    fin = _leaky_relu(y2 + r)

    fin2 = jnp.stack(
        [jnp.concatenate([fin[:, q * seg + d * dblk:q * seg + (d + 1) * dblk]
                          for q in range(4)], axis=1)
         for d in range(nd)], axis=1)
    res = jnp.dot(fin2.reshape(cout * nd, hwb).astype(vperm_ref.dtype),
                  uperm_ref[...], preferred_element_type=jnp.float32)
    o_ref[blk] = res.reshape(cout, nd, hwb)


def kernel(inp, skip, wt_mat, w1_mats, w2_mats, w3_mat, g1, b1, g2, b2, g3, b3):
    B, Cin, D, H, W = inp.shape
    Cout = skip.shape[1]
    Do, Ho, Wo = 2 * D, 2 * H, 2 * W
    S = Do * Ho * Wo
    seg = Do * H * W
    pad = H * W

    wt2 = wt_mat.reshape(Cout, 8, Cin).transpose(1, 0, 2).reshape(8 * Cout, Cin)

    hwb = Ho * Wo
    c = jnp.arange(hwb)
    qh, qw = (c // pad) // 2, (c // pad) % 2
    r = c % pad
    hp, wp = r // W, r % W
    row = (2 * hp + qh) * Wo + 2 * wp + qw
    vperm = (row[:, None] == jnp.arange(hwb)[None, :]).astype(jnp.bfloat16).T
    uperm = vperm.T

    body = functools.partial(_fused_body, seg=seg, pad=pad, wlen=W, hlen=H)
    out = pl.pallas_call(
        body,
        out_shape=jax.ShapeDtypeStruct((B, Cout, Do, hwb), jnp.float32),
        grid=(B // 4,),
        in_specs=[pl.BlockSpec((4, Cin, D * H * W), lambda b: (b, 0, 0)),
                  pl.BlockSpec((4, Cout, Do, hwb), lambda b: (b, 0, 0, 0)),
                  _rep(vperm), _rep(uperm), _rep(wt2),
                  _rep(w1_mats), _rep(g1), _rep(b1),
                  _rep(w3_mat), _rep(g3), _rep(b3),
                  _rep(w2_mats), _rep(g2), _rep(b2)],
        out_specs=pl.BlockSpec((4, Cout, Do, hwb), lambda b: (b, 0, 0, 0)),
        scratch_shapes=[
            pltpu.VMEM((9 * 2 * Cout, 4 * (seg + 2 * pad)), jnp.bfloat16),
            pltpu.VMEM((9 * Cout, 4 * (seg + 2 * pad)), jnp.bfloat16)],
        compiler_params=pltpu.CompilerParams(
            dimension_semantics=("arbitrary",),
            vmem_limit_bytes=48 * 1024 * 1024),
    )(inp.reshape(B, Cin, D * H * W), skip.reshape(B, Cout, Do, hwb),
      vperm, uperm, wt2,
      w1_mats.astype(jnp.bfloat16), g1, b1, w3_mat, g3, b3,
      w2_mats.astype(jnp.bfloat16), g2, b2)

    return out.reshape(B, Cout, Do, Ho, Wo)


# speedup vs baseline: 1.0120x; 1.0120x over previous
import functools

import jax
import jax.numpy as jnp
from jax.experimental import pallas as pl
from jax.experimental.pallas import tpu as pltpu

IN_EPS = 1e-5
NEG_SLOPE = 0.01


def _instance_norm(y, gamma, beta):
    mu = jnp.mean(y, axis=-1, keepdims=True)
    var = jnp.mean((y - mu) ** 2, axis=-1, keepdims=True)
    return (y - mu) * jax.lax.rsqrt(var + IN_EPS) * gamma + beta


def _leaky_relu(y):
    return jnp.where(y > 0, y, NEG_SLOPE * y)


def _rep(a):
    return pl.BlockSpec(a.shape, lambda b, _n=a.ndim: (0,) * _n)


def _hw_masks(seg, hlen, wlen):
    l = jax.lax.broadcasted_iota(jnp.int32, (1, seg), 1)
    hw = l % (hlen * wlen)
    hv = hw // wlen
    wv = hw % wlen
    masks = {}
    for sh in (-1, 0, 1):
        for sw in (-1, 0, 1):
            if sh == 0 and sw == 0:
                masks[(sh, sw)] = None
                continue
            valid = ((hv + sh >= 0) & (hv + sh < hlen)
                     & (wv + sw >= 0) & (wv + sw < wlen))
            masks[(sh, sw)] = valid.astype(jnp.float32)
    return masks


def _taps_to_scratch(x, p_s, masks, *, seg, pad, wlen, rows):
    segp = seg + 2 * pad
    slots = {}
    for kh in range(3):
        for kw in range(3):
            t = kh * 3 + kw
            for qh in range(2):
                for qw in range(2):
                    q = qh * 2 + qw
                    qsrc = ((qh + kh - 1) % 2) * 2 + ((qw + kw - 1) % 2)
                    sh = (qh + kh - 1) // 2
                    sw = (qw + kw - 1) // 2
                    slots.setdefault((qsrc, sh, sw), []).append((t, q))
    for (qsrc, sh, sw), dests in slots.items():
        src = x[:, qsrc * seg:(qsrc + 1) * seg]
        off = sh * wlen + sw
        if off:
            src = pltpu.roll(src, (-off) % seg, axis=1)
        m = masks[(sh, sw)]
        if m is not None:
            src = src * m
        srcb = src.astype(p_s.dtype)
        for (t, q) in dests:
            c0 = q * segp + pad
            p_s[t * rows:(t + 1) * rows, c0:c0 + seg] = srcb


def _conv3_hybrid(p_s, w_ref, *, seg, pad):
    segp = seg + 2 * pad
    wide = 3 * segp + seg
    acc = None
    for kd in range(3):
        d = jnp.dot(w_ref[kd], p_s[:, kd * pad:kd * pad + wide],
                    preferred_element_type=jnp.float32)
        acc = d if acc is None else acc + d
    return jnp.concatenate(
        [acc[:, q * segp:q * segp + seg] for q in range(4)], axis=1)


def _zero_pads(p_s, *, seg, pad):
    segp = seg + 2 * pad
    z = jnp.zeros((p_s.shape[0], pad), p_s.dtype)
    for q in range(4):
        p_s[:, q * segp:q * segp + pad] = z
        p_s[:, q * segp + pad + seg:(q + 1) * segp] = z


def _fused_body(inp_ref, skip_ref, vperm_ref, uperm_ref, wt_ref, w1_ref, g1_ref, b1_ref,
                w3_ref, g3_ref, b3_ref, w2_ref, g2_ref, b2_ref,
                o_ref, p1_s, p2_s, *, seg, pad, wlen, hlen):
    @pl.when(pl.program_id(0) == 0)
    def _():
        _zero_pads(p1_s, seg=seg, pad=pad)
        _zero_pads(p2_s, seg=seg, pad=pad)
    for blk in range(o_ref.shape[0]):
        _one_batch(blk, inp_ref, skip_ref, vperm_ref, uperm_ref, wt_ref,
                   w1_ref, g1_ref, b1_ref, w3_ref, g3_ref, b3_ref,
                   w2_ref, g2_ref, b2_ref, o_ref, p1_s, p2_s,
                   seg=seg, pad=pad, wlen=wlen, hlen=hlen)


def _one_batch(blk, inp_ref, skip_ref, vperm_ref, uperm_ref, wt_ref, w1_ref,
               g1_ref, b1_ref, w3_ref, g3_ref, b3_ref, w2_ref, g2_ref, b2_ref,
               o_ref, p1_s, p2_s, *, seg, pad, wlen, hlen):
    cout = o_ref.shape[1]
    dblk = pad
    nd = seg // dblk
    hwb = 4 * dblk

    up2 = jnp.dot(wt_ref[...], inp_ref[blk].astype(wt_ref.dtype),
                  preferred_element_type=jnp.float32)

    segs = []
    for q in range(4):
        a = up2[q * cout:(q + 1) * cout]
        b = up2[(4 + q) * cout:(5 + q) * cout]
        for dp in range(nd // 2):
            segs.append(a[:, dp * dblk:(dp + 1) * dblk])
            segs.append(b[:, dp * dblk:(dp + 1) * dblk])
    x_up = jnp.concatenate(segs, axis=1)

    skipm = skip_ref[blk].reshape(cout * nd, hwb).astype(vperm_ref.dtype)
    permres = jnp.dot(skipm, vperm_ref[...],
                      preferred_element_type=jnp.float32)
    perm3 = permres.reshape(cout, nd, hwb)
    x_skip = jnp.concatenate(
        [perm3[:, d, q * dblk:(q + 1) * dblk]
         for q in range(4) for d in range(nd)], axis=1)
    x = jnp.concatenate([x_up, x_skip], axis=0)

    masks = _hw_masks(seg, hlen, wlen)

    _taps_to_scratch(x, p1_s, masks, seg=seg, pad=pad, wlen=wlen,
                     rows=2 * cout)
    y1 = _leaky_relu(_instance_norm(
        _conv3_hybrid(p1_s, w1_ref, seg=seg, pad=pad),
        g1_ref[...], b1_ref[...]))

    r = _instance_norm(
        jnp.dot(w3_ref[...], x.astype(w3_ref.dtype),
                preferred_element_type=jnp.float32),
        g3_ref[...], b3_ref[...])

    _taps_to_scratch(y1, p2_s, masks, seg=seg, pad=pad, wlen=wlen, rows=cout)
    y2 = _instance_norm(
        _conv3_hybrid(p2_s, w2_ref, seg=seg, pad=pad),
        g2_ref[...], b2_ref[...])
    fin = _leaky_relu(y2 + r)

    fin2 = jnp.stack(
        [jnp.concatenate([fin[:, q * seg + d * dblk:q * seg + (d + 1) * dblk]
                          for q in range(4)], axis=1)
         for d in range(nd)], axis=1)
    res = jnp.dot(fin2.reshape(cout * nd, hwb).astype(vperm_ref.dtype),
                  uperm_ref[...], preferred_element_type=jnp.float32)
    o_ref[blk] = res.reshape(cout, nd, hwb)


def kernel(inp, skip, wt_mat, w1_mats, w2_mats, w3_mat, g1, b1, g2, b2, g3, b3):
    B, Cin, D, H, W = inp.shape
    Cout = skip.shape[1]
    Do, Ho, Wo = 2 * D, 2 * H, 2 * W
    S = Do * Ho * Wo
    seg = Do * H * W
    pad = H * W

    wt2 = wt_mat.reshape(Cout, 8, Cin).transpose(1, 0, 2).reshape(8 * Cout, Cin)

    hwb = Ho * Wo
    c = jnp.arange(hwb)
    qh, qw = (c // pad) // 2, (c // pad) % 2
    r = c % pad
    hp, wp = r // W, r % W
    row = (2 * hp + qh) * Wo + 2 * wp + qw
    vperm = (row[:, None] == jnp.arange(hwb)[None, :]).astype(jnp.bfloat16).T
    uperm = vperm.T

    body = functools.partial(_fused_body, seg=seg, pad=pad, wlen=W, hlen=H)
    out = pl.pallas_call(
        body,
        out_shape=jax.ShapeDtypeStruct((B, Cout, Do, hwb), jnp.float32),
        grid=(B // 2,),
        in_specs=[pl.BlockSpec((2, Cin, D * H * W), lambda b: (b, 0, 0)),
                  pl.BlockSpec((2, Cout, Do, hwb), lambda b: (b, 0, 0, 0)),
                  _rep(vperm), _rep(uperm), _rep(wt2),
                  _rep(w1_mats), _rep(g1), _rep(b1),
                  _rep(w3_mat), _rep(g3), _rep(b3),
                  _rep(w2_mats), _rep(g2), _rep(b2)],
        out_specs=pl.BlockSpec((2, Cout, Do, hwb), lambda b: (b, 0, 0, 0)),
        scratch_shapes=[
            pltpu.VMEM((9 * 2 * Cout, 4 * (seg + 2 * pad)), jnp.bfloat16),
            pltpu.VMEM((9 * Cout, 4 * (seg + 2 * pad)), jnp.bfloat16)],
        compiler_params=pltpu.CompilerParams(
            dimension_semantics=("arbitrary",),
            vmem_limit_bytes=48 * 1024 * 1024),
    )(inp.reshape(B, Cin, D * H * W), skip.reshape(B, Cout, Do, hwb),
      vperm, uperm, wt2.astype(jnp.bfloat16),
      w1_mats.astype(jnp.bfloat16), g1, b1,
      w3_mat.astype(jnp.bfloat16), g3, b3,
      w2_mats.astype(jnp.bfloat16), g2, b2)

    return out.reshape(B, Cout, Do, Ho, Wo)
